# Initial kernel scaffold; baseline (speedup 1.0000x reference)
import jax
import jax.numpy as jnp
from jax.experimental import pallas as pl


def kernel(x_bug, x_user, edge_index_bug_user, edge_index_user_bug, l1_proj_bug_w, l1_proj_bug_b, l1_proj_user_w, l1_proj_user_b, l1_asrc_bu, l1_adst_bu, l1_asrc_ub, l1_adst_ub, l1_klin_w, l1_klin_b, l1_q, l2_proj_bug_w, l2_proj_bug_b, l2_proj_user_w, l2_proj_user_b, l2_asrc_bu, l2_adst_bu, l2_asrc_ub, l2_adst_ub, l2_klin_w, l2_klin_b, l2_q, cls_w, cls_b):
    raise NotImplementedError("write your pallas kernel here")


# speedup vs baseline: 26.1974x; 26.1974x over previous
import functools

import jax
import jax.numpy as jnp
from jax import lax
from jax.experimental import pallas as pl
from jax.experimental.pallas import tpu as pltpu
from jax.experimental.pallas import tpu_sc as plsc

N = 25000
NPAD = 25088
E = 300000
EPAD = 307200
D = 128
NEG = 0.2
NC, NS, L = 2, 16, 16
NT = NC * NS
DPT = NPAD // NT
EPT = EPAD // NT
HALF = NPAD // 2
ACC_ROWS = 12672

_f32 = jnp.float32
_i32 = jnp.int32

_GDN = lax.GatherDimensionNumbers(
    offset_dims=(), collapsed_slice_dims=(0,), start_index_map=(0,))


def _bcast_lane(v, k):
    idx = jnp.full((L, 1), k, _i32)
    return lax.gather(v, idx, _GDN, (1,),
                      mode=lax.GatherScatterMode.PROMISE_IN_BOUNDS)


def _sc_mesh():
    return plsc.VectorSubcoreMesh(core_axis_name="c", subcore_axis_name="s")


def _wid():
    return lax.axis_index("c") * NS + lax.axis_index("s")


def _sc_alpha(src, dst, asrc_f, adst_f, H):
    def body(src_h, dst_h, asrc_h, adst_h, alpha_h, idx_v, tmp_v, tbl_v):
        base = _wid() * EPT

        pltpu.sync_copy(asrc_h, tbl_v)
        pltpu.sync_copy(src_h.at[pl.ds(base, EPT)], idx_v)

        def pass1(i, _):
            s16 = idx_v[pl.ds(i * L, L)]
            for h in range(H):
                g = plsc.load_gather(tbl_v, [s16 + h * NPAD])
                tmp_v[pl.ds(h * EPT + i * L, L)] = g
            return 0

        lax.fori_loop(0, EPT // L, pass1, 0)

        pltpu.sync_copy(adst_h, tbl_v)
        pltpu.sync_copy(dst_h.at[pl.ds(base, EPT)], idx_v)

        def pass2(i, _):
            d16 = idx_v[pl.ds(i * L, L)]
            for h in range(H):
                g = plsc.load_gather(tbl_v, [d16 + h * NPAD])
                al = tmp_v[pl.ds(h * EPT + i * L, L)] + g
                al = jnp.where(al >= 0.0, al, NEG * al)
                tmp_v[pl.ds(h * EPT + i * L, L)] = al
            return 0

        lax.fori_loop(0, EPT // L, pass2, 0)

        for h in range(H):
            pltpu.sync_copy(tmp_v.at[pl.ds(h * EPT, EPT)],
                            alpha_h.at[pl.ds(h * EPAD + base, EPT)])

    fn = pl.kernel(
        body,
        out_type=jax.ShapeDtypeStruct((H * EPAD,), _f32),
        mesh=_sc_mesh(),
        compiler_params=pltpu.CompilerParams(needs_layout_passes=False),
        scratch_types=[
            pltpu.VMEM((EPT,), _i32),
            pltpu.VMEM((H * EPT,), _f32),
            pltpu.VMEM((H * NPAD,), _f32),
        ],
    )
    return fn(src, dst, asrc_f, adst_f)


def _sc_maxden(dst, alpha, H):
    CH = 4800

    def body(dst_h, alpha_h, pm_h, ps_h, dbuf_v, abuf_v, mt_v, st_v):
        w = _wid()
        base = w * EPT

        def zinit(i, _):
            mt_v[pl.ds(i * L, L)] = jnp.full((L,), -1e30, _f32)
            st_v[pl.ds(i * L, L)] = jnp.zeros((L,), _f32)
            return 0

        lax.fori_loop(0, (H * NPAD) // L, zinit, 0)

        def scan(c, phase):
            pltpu.sync_copy(dst_h.at[pl.ds(base + c * CH, CH)], dbuf_v)
            for h in range(H):
                pltpu.sync_copy(
                    alpha_h.at[pl.ds(h * EPAD + base + c * CH, CH)],
                    abuf_v.at[pl.ds(h * CH, CH)])

            def vec(i, _):
                d16 = dbuf_v[pl.ds(i * L, L)]
                for h in range(H):
                    a16 = abuf_v[pl.ds(h * CH + i * L, L)]
                    gi = d16 * H + h
                    if phase == 0:
                        cur = plsc.load_gather(mt_v, [gi])
                        plsc.store_scatter(mt_v, [gi], jnp.maximum(cur, a16))
                    else:
                        m16 = plsc.load_gather(mt_v, [gi])
                        ex = jnp.exp(a16 - m16)
                        plsc.addupdate_scatter(st_v, [gi], ex)
                return 0

            lax.fori_loop(0, CH // L, vec, 0)
            return 0

        lax.fori_loop(0, EPT // CH, lambda c, _: scan(c, 0), 0)
        lax.fori_loop(0, EPT // CH, lambda c, _: scan(c, 1), 0)

        pltpu.sync_copy(mt_v, pm_h.at[pl.ds(w * (H * NPAD), H * NPAD)])
        pltpu.sync_copy(st_v, ps_h.at[pl.ds(w * (H * NPAD), H * NPAD)])

    fn = pl.kernel(
        body,
        out_type=(jax.ShapeDtypeStruct((NT * H * NPAD,), _f32),
                  jax.ShapeDtypeStruct((NT * H * NPAD,), _f32)),
        mesh=_sc_mesh(),
        compiler_params=pltpu.CompilerParams(needs_layout_passes=False),
        scratch_types=[
            pltpu.VMEM((CH,), _i32),
            pltpu.VMEM((H * CH,), _f32),
            pltpu.VMEM((H * NPAD,), _f32),
            pltpu.VMEM((H * NPAD,), _f32),
        ],
    )
    return fn(dst, alpha)


def _sc_merge(pm, ps, H):
    W = H * DPT

    def body(pm_h, ps_h, m_out, s_out, pma_v, psa_v, macc_v, sacc_v, sem):
        w = _wid()
        lo = w * W

        descs = []
        for p in range(NT):
            descs.append(pltpu.async_copy(
                pm_h.at[pl.ds(p * (H * NPAD) + lo, W)],
                pma_v.at[pl.ds(p * W, W)], sem))
            descs.append(pltpu.async_copy(
                ps_h.at[pl.ds(p * (H * NPAD) + lo, W)],
                psa_v.at[pl.ds(p * W, W)], sem))
        for d in descs:
            d.wait()

        def zinit(i, _):
            macc_v[pl.ds(i * L, L)] = jnp.full((L,), -1e30, _f32)
            sacc_v[pl.ds(i * L, L)] = jnp.zeros((L,), _f32)
            return 0

        lax.fori_loop(0, W // L, zinit, 0)

        def mx(i, _):
            for p in range(NT):
                v = pma_v[pl.ds(p * W + i * L, L)]
                macc_v[pl.ds(i * L, L)] = jnp.maximum(
                    macc_v[pl.ds(i * L, L)], v)
            return 0

        lax.fori_loop(0, W // L, mx, 0)

        def sm(i, _):
            m16 = macc_v[pl.ds(i * L, L)]
            acc = jnp.zeros((L,), _f32)
            for p in range(NT):
                mp = pma_v[pl.ds(p * W + i * L, L)]
                sp = psa_v[pl.ds(p * W + i * L, L)]
                acc = acc + jnp.exp(mp - m16) * sp
            sacc_v[pl.ds(i * L, L)] = acc
            return 0

        lax.fori_loop(0, W // L, sm, 0)

        pltpu.sync_copy(macc_v, m_out.at[pl.ds(lo, W)])
        pltpu.sync_copy(sacc_v, s_out.at[pl.ds(lo, W)])

    fn = pl.kernel(
        body,
        out_type=(jax.ShapeDtypeStruct((H * NPAD,), _f32),
                  jax.ShapeDtypeStruct((H * NPAD,), _f32)),
        mesh=_sc_mesh(),
        compiler_params=pltpu.CompilerParams(needs_layout_passes=False),
        scratch_types=[
            pltpu.VMEM((NT * W,), _f32),
            pltpu.VMEM((NT * W,), _f32),
            pltpu.VMEM((W,), _f32),
            pltpu.VMEM((W,), _f32),
            pltpu.SemaphoreType.DMA,
        ],
    )
    return fn(pm, ps)


def _sc_attn(dst, alpha, c_tab, H):
    def body(dst_h, alpha_h, c_h, attn_h, idx_v, tmp_v, tbl_v):
        base = _wid() * EPT

        pltpu.sync_copy(c_h, tbl_v)
        pltpu.sync_copy(dst_h.at[pl.ds(base, EPT)], idx_v)
        for h in range(H):
            pltpu.sync_copy(alpha_h.at[pl.ds(h * EPAD + base, EPT)],
                            tmp_v.at[pl.ds(h * EPT, EPT)])

        def vec(i, _):
            d16 = idx_v[pl.ds(i * L, L)]
            for h in range(H):
                cg = plsc.load_gather(tbl_v, [d16 * H + h])
                a16 = tmp_v[pl.ds(h * EPT + i * L, L)]
                tmp_v[pl.ds(h * EPT + i * L, L)] = jnp.exp(a16 - cg)
            return 0

        lax.fori_loop(0, EPT // L, vec, 0)

        for h in range(H):
            pltpu.sync_copy(tmp_v.at[pl.ds(h * EPT, EPT)],
                            attn_h.at[pl.ds(h * EPAD + base, EPT)])

    fn = pl.kernel(
        body,
        out_type=jax.ShapeDtypeStruct((H * EPAD,), _f32),
        mesh=_sc_mesh(),
        compiler_params=pltpu.CompilerParams(needs_layout_passes=False),
        scratch_types=[
            pltpu.VMEM((EPT,), _i32),
            pltpu.VMEM((H * EPT,), _f32),
            pltpu.VMEM((H * NPAD,), _f32),
        ],
    )
    return fn(dst, alpha, c_tab)


def _sc_agg(src, dst, attn, xsrc, H):
    CH = 1280
    BK = 128
    EPS = EPAD // NS
    HD = D // H

    def body(src_h, dst_h, attn_h, xsrc_h, agg_h,
             sbuf_v, dbuf_v, abuf_v, gidx_v, didx_v, atb_v,
             rows_v, zrow_v, acc_sh, sem):
        cid = lax.axis_index("c")
        sid = lax.axis_index("s")

        for r in range(8):
            for j in range(8):
                zrow_v[r, pl.ds(j * L, L)] = jnp.zeros((L,), _f32)

        def zacc(i, _):
            pltpu.sync_copy(zrow_v,
                            acc_sh.at[pl.ds(sid * (ACC_ROWS // NS) + i * 8, 8)])
            return 0

        lax.fori_loop(0, (ACC_ROWS // NS) // 8, zacc, 0)
        plsc.subcore_barrier()

        half_lo = cid * HALF

        def chunk(ci, _):
            base = sid * EPS + ci * CH
            pltpu.sync_copy(src_h.at[pl.ds(base, CH)], sbuf_v)
            pltpu.sync_copy(dst_h.at[pl.ds(base, CH)], dbuf_v)
            for h in range(H):
                pltpu.sync_copy(attn_h.at[pl.ds(h * EPAD + base, CH)],
                                abuf_v.at[pl.ds(h * CH, CH)])

            def batch(b, _):
                off = b * BK
                for j in range(BK // L):
                    gidx_v[pl.ds(j * L, L)] = sbuf_v[pl.ds(off + j * L, L)]
                dma = pltpu.async_copy(xsrc_h.at[gidx_v], rows_v, sem)

                for j in range(BK // L):
                    d16 = dbuf_v[pl.ds(off + j * L, L)]
                    inhalf = (d16 >= half_lo) & (d16 < half_lo + HALF)
                    dl = jnp.where(inhalf, d16 - half_lo, HALF)
                    didx_v[pl.ds(j * L, L)] = dl
                    for h in range(H):
                        a16 = abuf_v[pl.ds(h * CH + off + j * L, L)]
                        atb_v[pl.ds(h * BK + j * L, L)] = jnp.where(
                            inhalf, a16, 0.0)

                dma.wait()

                for g in range(BK // L):
                    av = [atb_v[pl.ds(h * BK + g * L, L)] for h in range(H)]
                    for k in range(L):
                        e = g * L + k
                        sc = [_bcast_lane(av[h], k) for h in range(H)]
                        for j in range(D // L):
                            h = (j * L) // HD
                            rows_v[e, pl.ds(j * L, L)] = (
                                rows_v[e, pl.ds(j * L, L)] * sc[h])

                pltpu.sync_copy(rows_v, acc_sh.at[didx_v], add=True)
                return 0

            lax.fori_loop(0, CH // BK, batch, 0)
            return 0

        lax.fori_loop(0, EPS // CH, chunk, 0)
        plsc.subcore_barrier()

        ro = sid * (HALF // NS)
        pltpu.sync_copy(acc_sh.at[pl.ds(ro, HALF // NS)],
                        agg_h.at[pl.ds(cid * HALF + ro, HALF // NS)])

    fn = pl.kernel(
        body,
        out_type=jax.ShapeDtypeStruct((NPAD, D), _f32),
        mesh=_sc_mesh(),
        compiler_params=pltpu.CompilerParams(needs_layout_passes=False),
        scratch_types=[
            pltpu.VMEM((CH,), _i32),
            pltpu.VMEM((CH,), _i32),
            pltpu.VMEM((H * CH,), _f32),
            pltpu.VMEM((BK,), _i32),
            pltpu.VMEM((BK,), _i32),
            pltpu.VMEM((H * BK,), _f32),
            pltpu.VMEM((BK, D), _f32),
            pltpu.VMEM((8, D), _f32),
            pltpu.VMEM_SHARED((ACC_ROWS, D), _f32),
            pltpu.SemaphoreType.DMA,
        ],
    )
    return fn(src, dst, attn, xsrc)


def _tc_proj(x, w, b8, s, relu_in):
    RB = 512

    def body(x_ref, w_ref, b_ref, s_ref, y_ref, a_ref):
        xb = x_ref[...]
        if relu_in:
            xb = jnp.maximum(xb, 0.0)
        y = jnp.dot(xb, w_ref[...], preferred_element_type=_f32) + b_ref[0:1]
        y_ref[...] = y
        a_ref[...] = jnp.dot(y, s_ref[...], preferred_element_type=_f32,
                             precision=lax.Precision.HIGHEST)

    return pl.pallas_call(
        body,
        grid=(NPAD // RB,),
        in_specs=[
            pl.BlockSpec((RB, D), lambda i: (i, 0)),
            pl.BlockSpec((D, D), lambda i: (0, 0)),
            pl.BlockSpec((8, D), lambda i: (0, 0)),
            pl.BlockSpec((D, D), lambda i: (0, 0)),
        ],
        out_specs=[
            pl.BlockSpec((RB, D), lambda i: (i, 0)),
            pl.BlockSpec((RB, D), lambda i: (i, 0)),
        ],
        out_shape=[
            jax.ShapeDtypeStruct((NPAD, D), _f32),
            jax.ShapeDtypeStruct((NPAD, D), _f32),
        ],
    )(x, w, b8, s)


def _tc_logc(m, s):
    R = m.shape[0] // D

    def body(m_ref, s_ref, c_ref):
        c_ref[...] = m_ref[...] + jnp.log(s_ref[...] + 1e-16)

    return pl.pallas_call(
        body,
        out_shape=jax.ShapeDtypeStruct((R, D), _f32),
    )(m.reshape(R, D), s.reshape(R, D)).reshape(-1)


def _tc_cls(x, w, b8):
    RB = 200
    C = 1000

    def body(x_ref, w_ref, b_ref, o_ref):
        xb = jnp.maximum(x_ref[...], 0.0)
        o_ref[...] = jnp.dot(xb, w_ref[...],
                             preferred_element_type=_f32) + b_ref[0:1]

    return pl.pallas_call(
        body,
        grid=(N // RB,),
        in_specs=[
            pl.BlockSpec((RB, D), lambda i: (i, 0)),
            pl.BlockSpec((D, C), lambda i: (0, 0)),
            pl.BlockSpec((8, C), lambda i: (0, 0)),
        ],
        out_specs=pl.BlockSpec((RB, C), lambda i: (i, 0)),
        out_shape=jax.ShapeDtypeStruct((N, C), _f32),
    )(x, w, b8)


def _smat(cols):
    out = []
    for vec, h, nh in cols:
        hd = D // nh
        hid = jnp.arange(D) // hd
        out.append(vec * (hid == h).astype(_f32))
    a = jnp.stack(out, axis=1)
    return jnp.pad(a, ((0, 0), (0, D - a.shape[1])))


def _edge_pass(src, dst, asrc_f, adst_f, xsrc, H):
    alpha = _sc_alpha(src, dst, asrc_f, adst_f, H)
    pm, ps = _sc_maxden(dst, alpha, H)
    m, s = _sc_merge(pm, ps, H)
    c = _tc_logc(m, s)
    attn = _sc_attn(dst, alpha, c, H)
    return _sc_agg(src, dst, attn, xsrc, H)


def _b8(b):
    return jnp.broadcast_to(b[None, :], (8, b.shape[0]))


def kernel(x_bug, x_user, edge_index_bug_user, edge_index_user_bug,
           l1_proj_bug_w, l1_proj_bug_b, l1_proj_user_w, l1_proj_user_b,
           l1_asrc_bu, l1_adst_bu, l1_asrc_ub, l1_adst_ub,
           l1_klin_w, l1_klin_b, l1_q,
           l2_proj_bug_w, l2_proj_bug_b, l2_proj_user_w, l2_proj_user_b,
           l2_asrc_bu, l2_adst_bu, l2_asrc_ub, l2_adst_ub,
           l2_klin_w, l2_klin_b, l2_q,
           cls_w, cls_b):
    pe = EPAD - E
    src_bu = jnp.concatenate(
        [edge_index_bug_user[0], jnp.zeros((pe,), _i32)])
    dst_bu = jnp.concatenate(
        [edge_index_bug_user[1], jnp.full((pe,), NPAD - 1, _i32)])
    src_ub = jnp.concatenate(
        [edge_index_user_bug[0], jnp.zeros((pe,), _i32)])
    dst_ub = jnp.concatenate(
        [edge_index_user_bug[1], jnp.full((pe,), NPAD - 1, _i32)])

    xb = jnp.pad(x_bug, ((0, NPAD - N), (0, 0)))
    xu = jnp.pad(x_user, ((0, NPAD - N), (0, 0)))

    s_b = _smat([(l1_asrc_bu.reshape(-1), 0, 2), (l1_asrc_bu.reshape(-1), 1, 2),
                 (l1_adst_ub.reshape(-1), 0, 2), (l1_adst_ub.reshape(-1), 1, 2)])
    s_u = _smat([(l1_adst_bu.reshape(-1), 0, 2), (l1_adst_bu.reshape(-1), 1, 2),
                 (l1_asrc_ub.reshape(-1), 0, 2), (l1_asrc_ub.reshape(-1), 1, 2)])

    y_b, a_b = _tc_proj(xb, l1_proj_bug_w, _b8(l1_proj_bug_b), s_b, False)
    y_u, a_u = _tc_proj(xu, l1_proj_user_w, _b8(l1_proj_user_b), s_u, False)

    asrc_bu_f = a_b[:, 0:2].T.reshape(-1)
    adst_ub_f = a_b[:, 2:4].T.reshape(-1)
    adst_bu_f = a_u[:, 0:2].T.reshape(-1)
    asrc_ub_f = a_u[:, 2:4].T.reshape(-1)

    agg_user = _edge_pass(src_bu, dst_bu, asrc_bu_f, adst_bu_f, y_b, 2)
    agg_bug = _edge_pass(src_ub, dst_ub, asrc_ub_f, adst_ub_f, y_u, 2)

    s_u2 = _smat([(l2_asrc_ub.reshape(-1), 0, 1)])
    s_b2 = _smat([(l2_adst_ub.reshape(-1), 0, 1)])
    y_u2, a_u2 = _tc_proj(agg_user, l2_proj_user_w, _b8(l2_proj_user_b),
                          s_u2, True)
    y_b2, a_b2 = _tc_proj(agg_bug, l2_proj_bug_w, _b8(l2_proj_bug_b),
                          s_b2, True)

    asrc2_f = a_u2[:, 0].reshape(-1)
    adst2_f = a_b2[:, 0].reshape(-1)

    agg2 = _edge_pass(src_ub, dst_ub, asrc2_f, adst2_f, y_u2, 1)

    return _tc_cls(agg2, cls_w, _b8(cls_b))


# speedup vs baseline: 27.5705x; 1.0524x over previous
import functools

import jax
import jax.numpy as jnp
from jax import lax
from jax.experimental import pallas as pl
from jax.experimental.pallas import tpu as pltpu
from jax.experimental.pallas import tpu_sc as plsc

N = 25000
NPAD = 25088
E = 300000
EPAD = 307200
D = 128
NEG = 0.2
NC, NS, L = 2, 16, 16
NT = NC * NS
DPT = NPAD // NT
EPT = EPAD // NT
HALF = NPAD // 2
ACC_ROWS = 12672

_f32 = jnp.float32
_i32 = jnp.int32

_GDN = lax.GatherDimensionNumbers(
    offset_dims=(), collapsed_slice_dims=(0,), start_index_map=(0,))


def _bcast_lane(v, k):
    idx = jnp.full((L, 1), k, _i32)
    return lax.gather(v, idx, _GDN, (1,),
                      mode=lax.GatherScatterMode.PROMISE_IN_BOUNDS)


def _sc_mesh():
    return plsc.VectorSubcoreMesh(core_axis_name="c", subcore_axis_name="s")


def _wid():
    return lax.axis_index("c") * NS + lax.axis_index("s")


def _sc_alpha(src, dst, asrc_f, adst_f, H):
    def body(src_h, dst_h, asrc_h, adst_h, alpha_h, idx_v, tmp_v, tbl_v):
        base = _wid() * EPT

        pltpu.sync_copy(asrc_h, tbl_v)
        pltpu.sync_copy(src_h.at[pl.ds(base, EPT)], idx_v)

        def pass1(i, _):
            s16 = idx_v[pl.ds(i * L, L)]
            for h in range(H):
                g = plsc.load_gather(tbl_v, [s16 + h * NPAD])
                tmp_v[pl.ds(h * EPT + i * L, L)] = g
            return 0

        lax.fori_loop(0, EPT // L, pass1, 0)

        pltpu.sync_copy(adst_h, tbl_v)
        pltpu.sync_copy(dst_h.at[pl.ds(base, EPT)], idx_v)

        def pass2(i, _):
            d16 = idx_v[pl.ds(i * L, L)]
            for h in range(H):
                g = plsc.load_gather(tbl_v, [d16 + h * NPAD])
                al = tmp_v[pl.ds(h * EPT + i * L, L)] + g
                al = jnp.where(al >= 0.0, al, NEG * al)
                tmp_v[pl.ds(h * EPT + i * L, L)] = al
            return 0

        lax.fori_loop(0, EPT // L, pass2, 0)

        for h in range(H):
            pltpu.sync_copy(tmp_v.at[pl.ds(h * EPT, EPT)],
                            alpha_h.at[pl.ds(h * EPAD + base, EPT)])

    fn = pl.kernel(
        body,
        out_type=jax.ShapeDtypeStruct((H * EPAD,), _f32),
        mesh=_sc_mesh(),
        compiler_params=pltpu.CompilerParams(needs_layout_passes=False),
        scratch_types=[
            pltpu.VMEM((EPT,), _i32),
            pltpu.VMEM((H * EPT,), _f32),
            pltpu.VMEM((H * NPAD,), _f32),
        ],
    )
    return fn(src, dst, asrc_f, adst_f)


def _sc_maxden(dst, alpha, H):
    CH = 4800

    def body(dst_h, alpha_h, pm_h, ps_h, dbuf_v, abuf_v, mt_v, st_v):
        w = _wid()
        base = w * EPT

        def zinit(i, _):
            mt_v[pl.ds(i * L, L)] = jnp.full((L,), -1e30, _f32)
            st_v[pl.ds(i * L, L)] = jnp.zeros((L,), _f32)
            return 0

        lax.fori_loop(0, (H * NPAD) // L, zinit, 0)

        def scan(c, phase):
            pltpu.sync_copy(dst_h.at[pl.ds(base + c * CH, CH)], dbuf_v)
            for h in range(H):
                pltpu.sync_copy(
                    alpha_h.at[pl.ds(h * EPAD + base + c * CH, CH)],
                    abuf_v.at[pl.ds(h * CH, CH)])

            def vec(i, _):
                d16 = dbuf_v[pl.ds(i * L, L)]
                for h in range(H):
                    a16 = abuf_v[pl.ds(h * CH + i * L, L)]
                    gi = d16 * H + h
                    if phase == 0:
                        cur = plsc.load_gather(mt_v, [gi])
                        plsc.store_scatter(mt_v, [gi], jnp.maximum(cur, a16))
                    else:
                        m16 = plsc.load_gather(mt_v, [gi])
                        ex = jnp.exp(a16 - m16)
                        plsc.addupdate_scatter(st_v, [gi], ex)
                return 0

            lax.fori_loop(0, CH // L, vec, 0)
            return 0

        lax.fori_loop(0, EPT // CH, lambda c, _: scan(c, 0), 0)
        lax.fori_loop(0, EPT // CH, lambda c, _: scan(c, 1), 0)

        pltpu.sync_copy(mt_v, pm_h.at[pl.ds(w * (H * NPAD), H * NPAD)])
        pltpu.sync_copy(st_v, ps_h.at[pl.ds(w * (H * NPAD), H * NPAD)])

    fn = pl.kernel(
        body,
        out_type=(jax.ShapeDtypeStruct((NT * H * NPAD,), _f32),
                  jax.ShapeDtypeStruct((NT * H * NPAD,), _f32)),
        mesh=_sc_mesh(),
        compiler_params=pltpu.CompilerParams(needs_layout_passes=False),
        scratch_types=[
            pltpu.VMEM((CH,), _i32),
            pltpu.VMEM((H * CH,), _f32),
            pltpu.VMEM((H * NPAD,), _f32),
            pltpu.VMEM((H * NPAD,), _f32),
        ],
    )
    return fn(dst, alpha)


def _sc_merge(pm, ps, H):
    W = H * DPT

    def body(pm_h, ps_h, m_out, s_out, pma_v, psa_v, macc_v, sacc_v, sem):
        w = _wid()
        lo = w * W

        descs = []
        for p in range(NT):
            descs.append(pltpu.async_copy(
                pm_h.at[pl.ds(p * (H * NPAD) + lo, W)],
                pma_v.at[pl.ds(p * W, W)], sem))
            descs.append(pltpu.async_copy(
                ps_h.at[pl.ds(p * (H * NPAD) + lo, W)],
                psa_v.at[pl.ds(p * W, W)], sem))
        for d in descs:
            d.wait()

        def zinit(i, _):
            macc_v[pl.ds(i * L, L)] = jnp.full((L,), -1e30, _f32)
            sacc_v[pl.ds(i * L, L)] = jnp.zeros((L,), _f32)
            return 0

        lax.fori_loop(0, W // L, zinit, 0)

        def mx(i, _):
            for p in range(NT):
                v = pma_v[pl.ds(p * W + i * L, L)]
                macc_v[pl.ds(i * L, L)] = jnp.maximum(
                    macc_v[pl.ds(i * L, L)], v)
            return 0

        lax.fori_loop(0, W // L, mx, 0)

        def sm(i, _):
            m16 = macc_v[pl.ds(i * L, L)]
            acc = jnp.zeros((L,), _f32)
            for p in range(NT):
                mp = pma_v[pl.ds(p * W + i * L, L)]
                sp = psa_v[pl.ds(p * W + i * L, L)]
                acc = acc + jnp.exp(mp - m16) * sp
            sacc_v[pl.ds(i * L, L)] = acc
            return 0

        lax.fori_loop(0, W // L, sm, 0)

        pltpu.sync_copy(macc_v, m_out.at[pl.ds(lo, W)])
        pltpu.sync_copy(sacc_v, s_out.at[pl.ds(lo, W)])

    fn = pl.kernel(
        body,
        out_type=(jax.ShapeDtypeStruct((H * NPAD,), _f32),
                  jax.ShapeDtypeStruct((H * NPAD,), _f32)),
        mesh=_sc_mesh(),
        compiler_params=pltpu.CompilerParams(needs_layout_passes=False),
        scratch_types=[
            pltpu.VMEM((NT * W,), _f32),
            pltpu.VMEM((NT * W,), _f32),
            pltpu.VMEM((W,), _f32),
            pltpu.VMEM((W,), _f32),
            pltpu.SemaphoreType.DMA,
        ],
    )
    return fn(pm, ps)


def _sc_attn(dst, alpha, c_tab, H):
    def body(dst_h, alpha_h, c_h, attn_h, idx_v, tmp_v, tbl_v):
        base = _wid() * EPT

        pltpu.sync_copy(c_h, tbl_v)
        pltpu.sync_copy(dst_h.at[pl.ds(base, EPT)], idx_v)
        for h in range(H):
            pltpu.sync_copy(alpha_h.at[pl.ds(h * EPAD + base, EPT)],
                            tmp_v.at[pl.ds(h * EPT, EPT)])

        def vec(i, _):
            d16 = idx_v[pl.ds(i * L, L)]
            for h in range(H):
                cg = plsc.load_gather(tbl_v, [d16 * H + h])
                a16 = tmp_v[pl.ds(h * EPT + i * L, L)]
                tmp_v[pl.ds(h * EPT + i * L, L)] = jnp.exp(a16 - cg)
            return 0

        lax.fori_loop(0, EPT // L, vec, 0)

        for h in range(H):
            pltpu.sync_copy(tmp_v.at[pl.ds(h * EPT, EPT)],
                            attn_h.at[pl.ds(h * EPAD + base, EPT)])

    fn = pl.kernel(
        body,
        out_type=jax.ShapeDtypeStruct((H * EPAD,), _f32),
        mesh=_sc_mesh(),
        compiler_params=pltpu.CompilerParams(needs_layout_passes=False),
        scratch_types=[
            pltpu.VMEM((EPT,), _i32),
            pltpu.VMEM((H * EPT,), _f32),
            pltpu.VMEM((H * NPAD,), _f32),
        ],
    )
    return fn(dst, alpha, c_tab)


def _sc_agg(src, dst, attn, xsrc, H):
    CH = 800
    BK = 80
    NB = CH // BK
    EPS = EPAD // NS
    HD = D // H

    def body(src_h, dst_h, attn_h, xsrc_h, agg_h,
             sbuf_v, dbuf_v, abuf_v, gidx0_v, gidx1_v, didx0_v, didx1_v,
             atb0_v, atb1_v, rows0_v, rows1_v, zrow_v, acc_sh, gsem, ssem):
        cid = lax.axis_index("c")
        sid = lax.axis_index("s")
        gidx = (gidx0_v, gidx1_v)
        didx = (didx0_v, didx1_v)
        atb = (atb0_v, atb1_v)
        rows = (rows0_v, rows1_v)

        for r in range(8):
            for j in range(8):
                zrow_v[r, pl.ds(j * L, L)] = jnp.zeros((L,), _f32)

        def zacc(i, _):
            pltpu.sync_copy(zrow_v,
                            acc_sh.at[pl.ds(sid * (ACC_ROWS // NS) + i * 8, 8)])
            return 0

        lax.fori_loop(0, (ACC_ROWS // NS) // 8, zacc, 0)
        plsc.subcore_barrier()

        half_lo = cid * HALF

        def fill(b, pb):
            off = b * BK
            for j in range(BK // L):
                gidx[pb][pl.ds(j * L, L)] = sbuf_v[pl.ds(off + j * L, L)]
                d16 = dbuf_v[pl.ds(off + j * L, L)]
                inhalf = (d16 >= half_lo) & (d16 < half_lo + HALF)
                didx[pb][pl.ds(j * L, L)] = jnp.where(
                    inhalf, d16 - half_lo, HALF)
                for h in range(H):
                    a16 = abuf_v[pl.ds(h * CH + off + j * L, L)]
                    atb[pb][pl.ds(h * BK + j * L, L)] = jnp.where(
                        inhalf, a16, 0.0)

        def scale(pb):
            rr = rows[pb]
            aa = atb[pb]

            def grp(g, _):
                av = [aa[pl.ds(h * BK + g * L, L)] for h in range(H)]
                for k in range(L):
                    e = g * L + k
                    sc = [_bcast_lane(av[h], k) for h in range(H)]
                    for j in range(D // L):
                        h = (j * L) // HD
                        rr[e, pl.ds(j * L, L)] = rr[e, pl.ds(j * L, L)] * sc[h]
                return 0

            lax.fori_loop(0, BK // L, grp, 0)

        def chunk(ci, _):
            base = sid * EPS + ci * CH
            pltpu.sync_copy(src_h.at[pl.ds(base, CH)], sbuf_v)
            pltpu.sync_copy(dst_h.at[pl.ds(base, CH)], dbuf_v)
            for h in range(H):
                pltpu.sync_copy(attn_h.at[pl.ds(h * EPAD + base, CH)],
                                abuf_v.at[pl.ds(h * CH, CH)])

            gd = [None] * NB
            sd = [None] * NB
            for b in range(NB):
                pb = b & 1
                if b >= 2:
                    sd[b - 2].wait()
                fill(b, pb)
                gd[b] = pltpu.async_copy(xsrc_h.at[gidx[pb]], rows[pb], gsem)
                if b >= 1:
                    gd[b - 1].wait()
                    scale(1 - pb)
                    sd[b - 1] = pltpu.async_copy(
                        rows[1 - pb], acc_sh.at[didx[1 - pb]], ssem, add=True)
            gd[NB - 1].wait()
            scale((NB - 1) & 1)
            sd[NB - 1] = pltpu.async_copy(
                rows[(NB - 1) & 1], acc_sh.at[didx[(NB - 1) & 1]], ssem,
                add=True)
            sd[NB - 2].wait()
            sd[NB - 1].wait()
            return 0

        lax.fori_loop(0, EPS // CH, chunk, 0)
        plsc.subcore_barrier()

        ro = sid * (HALF // NS)
        pltpu.sync_copy(acc_sh.at[pl.ds(ro, HALF // NS)],
                        agg_h.at[pl.ds(cid * HALF + ro, HALF // NS)])

    fn = pl.kernel(
        body,
        out_type=jax.ShapeDtypeStruct((NPAD, D), _f32),
        mesh=_sc_mesh(),
        compiler_params=pltpu.CompilerParams(needs_layout_passes=False),
        scratch_types=[
            pltpu.VMEM((CH,), _i32),
            pltpu.VMEM((CH,), _i32),
            pltpu.VMEM((H * CH,), _f32),
            pltpu.VMEM((BK,), _i32),
            pltpu.VMEM((BK,), _i32),
            pltpu.VMEM((BK,), _i32),
            pltpu.VMEM((BK,), _i32),
            pltpu.VMEM((H * BK,), _f32),
            pltpu.VMEM((H * BK,), _f32),
            pltpu.VMEM((BK, D), _f32),
            pltpu.VMEM((BK, D), _f32),
            pltpu.VMEM((8, D), _f32),
            pltpu.VMEM_SHARED((ACC_ROWS, D), _f32),
            pltpu.SemaphoreType.DMA,
            pltpu.SemaphoreType.DMA,
        ],
    )
    return fn(src, dst, attn, xsrc)


def _tc_proj(x, w, b8, s, relu_in):
    RB = 512

    def body(x_ref, w_ref, b_ref, s_ref, y_ref, a_ref):
        xb = x_ref[...]
        if relu_in:
            xb = jnp.maximum(xb, 0.0)
        y = jnp.dot(xb, w_ref[...], preferred_element_type=_f32) + b_ref[0:1]
        y_ref[...] = y
        a_ref[...] = jnp.dot(y, s_ref[...], preferred_element_type=_f32,
                             precision=lax.Precision.HIGHEST)

    return pl.pallas_call(
        body,
        grid=(NPAD // RB,),
        in_specs=[
            pl.BlockSpec((RB, D), lambda i: (i, 0)),
            pl.BlockSpec((D, D), lambda i: (0, 0)),
            pl.BlockSpec((8, D), lambda i: (0, 0)),
            pl.BlockSpec((D, D), lambda i: (0, 0)),
        ],
        out_specs=[
            pl.BlockSpec((RB, D), lambda i: (i, 0)),
            pl.BlockSpec((RB, D), lambda i: (i, 0)),
        ],
        out_shape=[
            jax.ShapeDtypeStruct((NPAD, D), _f32),
            jax.ShapeDtypeStruct((NPAD, D), _f32),
        ],
    )(x, w, b8, s)


def _tc_logc(m, s):
    R = m.shape[0] // D

    def body(m_ref, s_ref, c_ref):
        c_ref[...] = m_ref[...] + jnp.log(s_ref[...] + 1e-16)

    return pl.pallas_call(
        body,
        out_shape=jax.ShapeDtypeStruct((R, D), _f32),
    )(m.reshape(R, D), s.reshape(R, D)).reshape(-1)


def _tc_cls(x, w, b8):
    RB = 200
    C = 1000

    def body(x_ref, w_ref, b_ref, o_ref):
        xb = jnp.maximum(x_ref[...], 0.0)
        o_ref[...] = jnp.dot(xb, w_ref[...],
                             preferred_element_type=_f32) + b_ref[0:1]

    return pl.pallas_call(
        body,
        grid=(N // RB,),
        in_specs=[
            pl.BlockSpec((RB, D), lambda i: (i, 0)),
            pl.BlockSpec((D, C), lambda i: (0, 0)),
            pl.BlockSpec((8, C), lambda i: (0, 0)),
        ],
        out_specs=pl.BlockSpec((RB, C), lambda i: (i, 0)),
        out_shape=jax.ShapeDtypeStruct((N, C), _f32),
    )(x, w, b8)


def _smat(cols):
    out = []
    for vec, h, nh in cols:
        hd = D // nh
        hid = jnp.arange(D) // hd
        out.append(vec * (hid == h).astype(_f32))
    a = jnp.stack(out, axis=1)
    return jnp.pad(a, ((0, 0), (0, D - a.shape[1])))


def _edge_pass(src, dst, asrc_f, adst_f, xsrc, H):
    alpha = _sc_alpha(src, dst, asrc_f, adst_f, H)
    pm, ps = _sc_maxden(dst, alpha, H)
    m, s = _sc_merge(pm, ps, H)
    c = _tc_logc(m, s)
    attn = _sc_attn(dst, alpha, c, H)
    return _sc_agg(src, dst, attn, xsrc, H)


def _b8(b):
    return jnp.broadcast_to(b[None, :], (8, b.shape[0]))


def kernel(x_bug, x_user, edge_index_bug_user, edge_index_user_bug,
           l1_proj_bug_w, l1_proj_bug_b, l1_proj_user_w, l1_proj_user_b,
           l1_asrc_bu, l1_adst_bu, l1_asrc_ub, l1_adst_ub,
           l1_klin_w, l1_klin_b, l1_q,
           l2_proj_bug_w, l2_proj_bug_b, l2_proj_user_w, l2_proj_user_b,
           l2_asrc_bu, l2_adst_bu, l2_asrc_ub, l2_adst_ub,
           l2_klin_w, l2_klin_b, l2_q,
           cls_w, cls_b):
    pe = EPAD - E
    src_bu = jnp.concatenate(
        [edge_index_bug_user[0], jnp.zeros((pe,), _i32)])
    dst_bu = jnp.concatenate(
        [edge_index_bug_user[1], jnp.full((pe,), NPAD - 1, _i32)])
    src_ub = jnp.concatenate(
        [edge_index_user_bug[0], jnp.zeros((pe,), _i32)])
    dst_ub = jnp.concatenate(
        [edge_index_user_bug[1], jnp.full((pe,), NPAD - 1, _i32)])

    xb = jnp.pad(x_bug, ((0, NPAD - N), (0, 0)))
    xu = jnp.pad(x_user, ((0, NPAD - N), (0, 0)))

    s_b = _smat([(l1_asrc_bu.reshape(-1), 0, 2), (l1_asrc_bu.reshape(-1), 1, 2),
                 (l1_adst_ub.reshape(-1), 0, 2), (l1_adst_ub.reshape(-1), 1, 2)])
    s_u = _smat([(l1_adst_bu.reshape(-1), 0, 2), (l1_adst_bu.reshape(-1), 1, 2),
                 (l1_asrc_ub.reshape(-1), 0, 2), (l1_asrc_ub.reshape(-1), 1, 2)])

    y_b, a_b = _tc_proj(xb, l1_proj_bug_w, _b8(l1_proj_bug_b), s_b, False)
    y_u, a_u = _tc_proj(xu, l1_proj_user_w, _b8(l1_proj_user_b), s_u, False)

    asrc_bu_f = a_b[:, 0:2].T.reshape(-1)
    adst_ub_f = a_b[:, 2:4].T.reshape(-1)
    adst_bu_f = a_u[:, 0:2].T.reshape(-1)
    asrc_ub_f = a_u[:, 2:4].T.reshape(-1)

    agg_user = _edge_pass(src_bu, dst_bu, asrc_bu_f, adst_bu_f, y_b, 2)
    agg_bug = _edge_pass(src_ub, dst_ub, asrc_ub_f, adst_ub_f, y_u, 2)

    s_u2 = _smat([(l2_asrc_ub.reshape(-1), 0, 1)])
    s_b2 = _smat([(l2_adst_ub.reshape(-1), 0, 1)])
    y_u2, a_u2 = _tc_proj(agg_user, l2_proj_user_w, _b8(l2_proj_user_b),
                          s_u2, True)
    y_b2, a_b2 = _tc_proj(agg_bug, l2_proj_bug_w, _b8(l2_proj_bug_b),
                          s_b2, True)

    asrc2_f = a_u2[:, 0].reshape(-1)
    adst2_f = a_b2[:, 0].reshape(-1)

    agg2 = _edge_pass(src_ub, dst_ub, asrc2_f, adst2_f, y_u2, 1)

    return _tc_cls(agg2, cls_w, _b8(cls_b))
